# baseline (speedup 1.0000x reference)
import functools

import jax
import jax.numpy as jnp
from jax import lax
from jax.experimental import pallas as pl
from jax.experimental.pallas import tpu as pltpu
from jax.experimental.pallas import tpu_sc as plsc

B, N, D, H, W, C = 2, 6, 41, 16, 44, 64
P = B * N * D * H * W
HW = H * W
NCHUNK = N * D
PAD_ROWS = 6
NX = 200
NCELL = NX * NX
ACC_ROWS = NCELL + 16
ZERO_PER_TILE = ACC_ROWS // 16
OUT_PER_TILE = NCELL // 16
CHALF = C // 2


def _cells(intrinsics, pose):
    ds_vals = jnp.arange(4.0, 45.0, 1.0, dtype=jnp.float32)
    ogf_h, ogf_w = H * 16, W * 16
    ones = jnp.ones((D, H, W), jnp.float32)
    ds_b = ds_vals.reshape(D, 1, 1) * ones
    xs = jnp.linspace(0.0, ogf_w - 1.0, W, dtype=jnp.float32).reshape(1, 1, W) * ones
    ys = jnp.linspace(0.0, ogf_h - 1.0, H, dtype=jnp.float32).reshape(1, H, 1) * ones
    frustum = jnp.stack((xs, ys, ds_b), -1)
    rots = pose[..., :3, :3]
    trans = pose[..., :3, 3]
    pts = jnp.concatenate(
        [frustum[..., :2] * frustum[..., 2:3], frustum[..., 2:3]], -1)
    combine = rots @ jnp.linalg.inv(intrinsics)
    geom = (jnp.einsum('bnij,dhwj->bndhwi', combine, pts)
            + trans[:, :, None, None, None, :])
    gf = jax.lax.stop_gradient(geom.reshape(P, 3))
    gx = gf[:, 0] * 2.0 + 100.0
    gy = gf[:, 1] * 2.0 + 100.0
    gz = (gf[:, 2] - 0.0 + 20.0 / 2.0) / 20.0
    ix = gx.astype(jnp.int32)
    iy = gy.astype(jnp.int32)
    iz = gz.astype(jnp.int32)
    kept = ((ix >= 0) & (ix < NX) & (iy >= 0) & (iy < NX)
            & (iz >= 0) & (iz < 1))
    return jnp.where(kept, ix * NX + iy, NCELL)


def _scatter_body(x_hbm, cellp_hbm, zeros_hbm, out_hbm, data_v, idx_v, acc_sh):
    core = lax.axis_index("c")
    sub = lax.axis_index("s")
    for p in range(2):
        zr = sub * ZERO_PER_TILE
        pltpu.sync_copy(zeros_hbm.at[pl.ds(zr, ZERO_PER_TILE)],
                        acc_sh.at[pl.ds(zr, ZERO_PER_TILE)])
        plsc.subcore_barrier()

        @pl.loop(sub, NCHUNK, step=16)
        def _chunk(ci):
            n = ci // D
            d = ci - n * D
            pltpu.sync_copy(cellp_hbm.at[core * NCHUNK + ci], idx_v)
            pltpu.sync_copy(
                x_hbm.at[core, n, d, pl.ds(0, HW), pl.ds(p * CHALF, CHALF)],
                data_v.at[pl.ds(0, HW)])
            for j in range(PAD_ROWS):
                pltpu.sync_copy(data_v.at[pl.ds(j * 128, 128)],
                                acc_sh.at[idx_v.at[j]], add=True)

        plsc.subcore_barrier()
        orow = sub * OUT_PER_TILE
        pltpu.sync_copy(acc_sh.at[pl.ds(orow, OUT_PER_TILE)],
                        out_hbm.at[core, p, pl.ds(orow, OUT_PER_TILE)])
        plsc.subcore_barrier()


_scatter_call = functools.partial(
    pl.kernel,
    out_type=jax.ShapeDtypeStruct((B, 2, NCELL, CHALF), jnp.float32),
    mesh=plsc.VectorSubcoreMesh(core_axis_name="c", subcore_axis_name="s"),
    scratch_types=[
        pltpu.VMEM((PAD_ROWS * 128, CHALF), jnp.float32),
        pltpu.VMEM((PAD_ROWS, 128), jnp.int32),
        pltpu.VMEM_SHARED((ACC_ROWS, CHALF), jnp.float32),
    ],
    compiler_params=pltpu.CompilerParams(use_tc_tiling_on_sc=False),
)(_scatter_body)


def kernel(x, intrinsics, pose):
    cell = _cells(intrinsics, pose)
    cellp = jnp.concatenate(
        [cell.reshape(B * N * D, HW),
         jnp.full((B * N * D, PAD_ROWS * 128 - HW), NCELL, jnp.int32)],
        axis=1).reshape(B * N * D, PAD_ROWS, 128)

    zeros = jnp.zeros((ACC_ROWS, CHALF), jnp.float32)
    acc = _scatter_call(x.reshape(B, N, D, HW, C), cellp, zeros)

    o = acc.reshape(B, 2, NX, NX, CHALF)
    return o.transpose(0, 1, 4, 3, 2).reshape(B, C, NX, NX)
